# baseline (speedup 1.0000x reference)
import functools

import jax
import jax.numpy as jnp
from jax import lax
from jax.experimental import pallas as pl
from jax.experimental.pallas import tpu as pltpu
from jax.experimental.pallas import tpu_sc as plsc

NODES = 10000
EDGES = 320000
D = 128
H = 256
C = 10
G = 64

CHUNK = 128
NWORKERS = 32
CPW = 79
EPAD = NWORKERS * CPW * CHUNK
RPAD = 10112
RCHUNKS = RPAD // CHUNK
DUMMY_ROW = NODES

f32 = jnp.float32


@functools.lru_cache(maxsize=None)
def _make_edge_agg(width):
    mesh = plsc.VectorSubcoreMesh(core_axis_name="c", subcore_axis_name="s")

    @functools.partial(
        pl.kernel,
        mesh=mesh,
        out_type=jax.ShapeDtypeStruct((2, RPAD, width), f32),
        scratch_types=[
            pltpu.VMEM((2, CHUNK), jnp.int32),
            pltpu.VMEM((CHUNK, width), f32),
            pltpu.VMEM_SHARED((RPAD, width), f32),
            pltpu.SemaphoreType.DMA,
        ],
    )
    def agg(h_hbm, eidx_hbm, zeros_hbm, out_hbm, ebuf, rows0, acc, sem0):
        cid = lax.axis_index("c")
        sid = lax.axis_index("s")
        wid = sid * 2 + cid

        pltpu.sync_copy(zeros_hbm, rows0)

        def zero_body(k, carry):
            chunk = sid + k * 16

            @pl.when(chunk < RCHUNKS)
            def _():
                off = pl.multiple_of(chunk * CHUNK, CHUNK)
                pltpu.sync_copy(rows0, acc.at[pl.ds(off, CHUNK)])

            return carry

        lax.fori_loop(0, (RCHUNKS + 15) // 16, zero_body, 0)
        plsc.subcore_barrier()

        ebase = wid * CPW

        def edge_body(i, carry):
            pltpu.sync_copy(eidx_hbm.at[ebase + i], ebuf)
            pltpu.async_copy(h_hbm.at[ebuf.at[0]], rows0, sem0).wait()
            pltpu.sync_copy(rows0, acc.at[ebuf.at[1]], add=True)
            return carry

        lax.fori_loop(0, CPW, edge_body, 0)
        plsc.subcore_barrier()

        def out_body(k, carry):
            chunk = sid + k * 16

            @pl.when(chunk < RCHUNKS)
            def _():
                off = pl.multiple_of(chunk * CHUNK, CHUNK)
                pltpu.sync_copy(acc.at[pl.ds(off, CHUNK)], rows0)
                pltpu.sync_copy(rows0, out_hbm.at[cid, pl.ds(off, CHUNK)])

            return carry

        lax.fori_loop(0, (RCHUNKS + 15) // 16, out_body, 0)

    return agg


def _mlp_body(x_ref, w1_ref, b1_ref, w2_ref, b2_ref, o_ref):
    h = jnp.dot(x_ref[...], w1_ref[...], preferred_element_type=f32)
    h = jnp.maximum(h + b1_ref[...], 0.0)
    h = jnp.dot(h, w2_ref[...], preferred_element_type=f32)
    o_ref[...] = jnp.maximum(h + b2_ref[...], 0.0)


def _comb1_body(p_ref, h_ref, wrel_ref, brel_ref, wroot_ref, oa_ref, ob_ref):
    agg = p_ref[0, :NODES, :] + p_ref[1, :NODES, :]
    out = (jnp.dot(agg, wrel_ref[...], preferred_element_type=f32)
           + brel_ref[...]
           + jnp.dot(h_ref[...], wroot_ref[...], preferred_element_type=f32))
    out = jnp.maximum(out, 0.0)
    oa_ref[...] = out[:, :128]
    ob_ref[...] = out[:, 128:]


def _comb2_body(pa_ref, pb_ref, ha_ref, hb_ref, wrel_ref, brel_ref,
                wroot_ref, oa_ref, ob_ref):
    agg_a = pa_ref[0, :NODES, :] + pa_ref[1, :NODES, :]
    agg_b = pb_ref[0, :NODES, :] + pb_ref[1, :NODES, :]
    out = (jnp.dot(agg_a, wrel_ref[:128, :], preferred_element_type=f32)
           + jnp.dot(agg_b, wrel_ref[128:, :], preferred_element_type=f32)
           + brel_ref[...]
           + jnp.dot(ha_ref[...], wroot_ref[:128, :], preferred_element_type=f32)
           + jnp.dot(hb_ref[...], wroot_ref[128:, :], preferred_element_type=f32))
    out = jnp.maximum(out, 0.0)
    oa_ref[...] = out[:, :128]
    ob_ref[...] = out[:, 128:]


def _pre3_body(ha_ref, hb_ref, wrel_ref, wroot_ref, z_ref, r_ref):
    z_ref[...] = (jnp.dot(ha_ref[...], wrel_ref[:128, :], preferred_element_type=f32)
                  + jnp.dot(hb_ref[...], wrel_ref[128:, :], preferred_element_type=f32))
    r_ref[...] = (jnp.dot(ha_ref[...], wroot_ref[:128, :], preferred_element_type=f32)
                  + jnp.dot(hb_ref[...], wroot_ref[128:, :], preferred_element_type=f32))


def _final_body(p_ref, r_ref, brel_ref, batch_ref, o_ref):
    out3 = (p_ref[0, :NODES, :] + p_ref[1, :NODES, :]
            + r_ref[...] + brel_ref[...])
    gids = lax.broadcasted_iota(jnp.int32, (NODES, G), 1)
    onehot = (batch_ref[...] == gids).astype(f32)
    sums = lax.dot_general(onehot, out3, (((0,), (0,)), ((), ())),
                           preferred_element_type=f32)
    cnt = jnp.sum(onehot, axis=0)[:, None]
    pooled = sums / jnp.maximum(cnt, 1.0)
    col = lax.broadcasted_iota(jnp.int32, (G, 128), 1)
    masked = jnp.where(col < C, pooled, -jnp.inf)
    m = jnp.max(masked, axis=1, keepdims=True)
    ex = jnp.where(col < C, jnp.exp(masked - m), 0.0)
    lse = jnp.log(jnp.sum(ex, axis=1, keepdims=True))
    o_ref[...] = masked - m - lse


def _sds(shape):
    return jax.ShapeDtypeStruct(shape, f32)


def kernel(x, edge_index, batch, mlp_W1, mlp_b1, mlp_W2, mlp_b2,
           Wrel1, brel1, Wroot1, Wrel2, brel2, Wroot2, Wrel3, brel3, Wroot3):
    src = edge_index[0]
    dst = edge_index[1]
    npad = EPAD - EDGES
    src_p = jnp.concatenate([src, jnp.zeros((npad,), jnp.int32)])
    dst_p = jnp.concatenate([dst, jnp.full((npad,), DUMMY_ROW, jnp.int32)])
    eidx = jnp.stack([src_p.reshape(-1, CHUNK), dst_p.reshape(-1, CHUNK)], axis=1)
    zeros128 = jnp.zeros((CHUNK, 128), f32)

    h1 = pl.pallas_call(_mlp_body, out_shape=_sds((NODES, D)))(
        x, mlp_W1, mlp_b1.reshape(1, -1), mlp_W2, mlp_b2.reshape(1, -1))

    p1 = _make_edge_agg(128)(h1, eidx, zeros128)
    oa1, ob1 = pl.pallas_call(
        _comb1_body, out_shape=(_sds((NODES, 128)), _sds((NODES, 128))))(
        p1, h1, Wrel1, brel1.reshape(1, -1), Wroot1)

    p2a = _make_edge_agg(128)(oa1, eidx, zeros128)
    p2b = _make_edge_agg(128)(ob1, eidx, zeros128)
    oa2, ob2 = pl.pallas_call(
        _comb2_body, out_shape=(_sds((NODES, 128)), _sds((NODES, 128))))(
        p2a, p2b, oa1, ob1, Wrel2, brel2.reshape(1, -1), Wroot2)

    Wrel3p = jnp.pad(Wrel3, ((0, 0), (0, 128 - C)))
    Wroot3p = jnp.pad(Wroot3, ((0, 0), (0, 128 - C)))
    brel3p = jnp.pad(brel3, (0, 128 - C)).reshape(1, -1)
    z, r = pl.pallas_call(
        _pre3_body, out_shape=(_sds((NODES, 128)), _sds((NODES, 128))))(
        oa2, ob2, Wrel3p, Wroot3p)
    p3 = _make_edge_agg(128)(z, eidx, zeros128)

    out = pl.pallas_call(_final_body, out_shape=_sds((G, 128)))(
        p3, r, brel3p, batch.reshape(-1, 1))
    return out[:, :C]


# speedup vs baseline: 1.0905x; 1.0905x over previous
import functools

import jax
import jax.numpy as jnp
from jax import lax
from jax.experimental import pallas as pl
from jax.experimental.pallas import tpu as pltpu
from jax.experimental.pallas import tpu_sc as plsc

NODES = 10000
EDGES = 320000
D = 128
H = 256
C = 10
G = 64

CHUNK = 128
NWORKERS = 32
CPW = 79
EPAD = NWORKERS * CPW * CHUNK
RPAD = 10112
RCHUNKS = RPAD // CHUNK
DUMMY_ROW = NODES

f32 = jnp.float32


@functools.lru_cache(maxsize=None)
def _make_edge_agg(width):
    mesh = plsc.VectorSubcoreMesh(core_axis_name="c", subcore_axis_name="s")

    @functools.partial(
        pl.kernel,
        mesh=mesh,
        out_type=jax.ShapeDtypeStruct((2, RPAD, width), f32),
        scratch_types=[
            pltpu.VMEM((2, CHUNK), jnp.int32),
            pltpu.VMEM((2, CHUNK), jnp.int32),
            pltpu.VMEM((CHUNK, width), f32),
            pltpu.VMEM_SHARED((RPAD, width), f32),
            pltpu.SemaphoreType.DMA,
            pltpu.SemaphoreType.DMA,
        ],
    )
    def agg(h_hbm, eidx_hbm, zeros_hbm, out_hbm, ebufa, ebufb, rows0, acc,
            sem0, sem1):
        cid = lax.axis_index("c")
        sid = lax.axis_index("s")
        wid = sid * 2 + cid

        pltpu.sync_copy(zeros_hbm, rows0)

        def zero_body(k, carry):
            chunk = sid + k * 16

            @pl.when(chunk < RCHUNKS)
            def _():
                off = pl.multiple_of(chunk * CHUNK, CHUNK)
                pltpu.sync_copy(rows0, acc.at[pl.ds(off, CHUNK)])

            return carry

        lax.fori_loop(0, (RCHUNKS + 15) // 16, zero_body, 0)
        plsc.subcore_barrier()

        ebase = wid * CPW
        nrows = NWORKERS * CPW

        def chunk(ebuf):
            pltpu.async_copy(h_hbm.at[ebuf.at[0]], rows0, sem0).wait()
            pltpu.sync_copy(rows0, acc.at[ebuf.at[1]], add=True)

        pltpu.sync_copy(eidx_hbm.at[ebase], ebufa)

        def edge_body(k, carry):
            a = k * 2
            n1 = pltpu.async_copy(
                eidx_hbm.at[jnp.minimum(ebase + a + 1, nrows - 1)], ebufb,
                sem1)
            chunk(ebufa)
            n1.wait()
            n2 = pltpu.async_copy(
                eidx_hbm.at[jnp.minimum(ebase + a + 2, nrows - 1)], ebufa,
                sem1)
            chunk(ebufb)
            n2.wait()
            return carry

        lax.fori_loop(0, CPW // 2, edge_body, 0)
        if CPW % 2:
            chunk(ebufa)
        plsc.subcore_barrier()

        def out_body(k, carry):
            chunk = sid + k * 16

            @pl.when(chunk < RCHUNKS)
            def _():
                off = pl.multiple_of(chunk * CHUNK, CHUNK)
                pltpu.sync_copy(acc.at[pl.ds(off, CHUNK)], rows0)
                pltpu.sync_copy(rows0, out_hbm.at[cid, pl.ds(off, CHUNK)])

            return carry

        lax.fori_loop(0, (RCHUNKS + 15) // 16, out_body, 0)

    return agg


def _mlp_body(x_ref, w1_ref, b1_ref, w2_ref, b2_ref, o_ref):
    h = jnp.dot(x_ref[...], w1_ref[...], preferred_element_type=f32)
    h = jnp.maximum(h + b1_ref[...], 0.0)
    h = jnp.dot(h, w2_ref[...], preferred_element_type=f32)
    o_ref[...] = jnp.maximum(h + b2_ref[...], 0.0)


def _comb1_body(p_ref, h_ref, wrel_ref, brel_ref, wroot_ref, oa_ref, ob_ref):
    agg = p_ref[0, :NODES, :] + p_ref[1, :NODES, :]
    out = (jnp.dot(agg, wrel_ref[...], preferred_element_type=f32)
           + brel_ref[...]
           + jnp.dot(h_ref[...], wroot_ref[...], preferred_element_type=f32))
    out = jnp.maximum(out, 0.0)
    oa_ref[...] = out[:, :128]
    ob_ref[...] = out[:, 128:]


def _comb2_body(pa_ref, pb_ref, ha_ref, hb_ref, wrel_ref, brel_ref,
                wroot_ref, oa_ref, ob_ref):
    agg_a = pa_ref[0, :NODES, :] + pa_ref[1, :NODES, :]
    agg_b = pb_ref[0, :NODES, :] + pb_ref[1, :NODES, :]
    out = (jnp.dot(agg_a, wrel_ref[:128, :], preferred_element_type=f32)
           + jnp.dot(agg_b, wrel_ref[128:, :], preferred_element_type=f32)
           + brel_ref[...]
           + jnp.dot(ha_ref[...], wroot_ref[:128, :], preferred_element_type=f32)
           + jnp.dot(hb_ref[...], wroot_ref[128:, :], preferred_element_type=f32))
    out = jnp.maximum(out, 0.0)
    oa_ref[...] = out[:, :128]
    ob_ref[...] = out[:, 128:]


def _pre3_body(ha_ref, hb_ref, wrel_ref, wroot_ref, z_ref, r_ref):
    z_ref[...] = (jnp.dot(ha_ref[...], wrel_ref[:128, :], preferred_element_type=f32)
                  + jnp.dot(hb_ref[...], wrel_ref[128:, :], preferred_element_type=f32))
    r_ref[...] = (jnp.dot(ha_ref[...], wroot_ref[:128, :], preferred_element_type=f32)
                  + jnp.dot(hb_ref[...], wroot_ref[128:, :], preferred_element_type=f32))


def _final_body(p_ref, r_ref, brel_ref, batch_ref, o_ref):
    out3 = (p_ref[0, :NODES, :] + p_ref[1, :NODES, :]
            + r_ref[...] + brel_ref[...])
    gids = lax.broadcasted_iota(jnp.int32, (NODES, G), 1)
    onehot = (batch_ref[...] == gids).astype(f32)
    sums = lax.dot_general(onehot, out3, (((0,), (0,)), ((), ())),
                           preferred_element_type=f32)
    cnt = jnp.sum(onehot, axis=0)[:, None]
    pooled = sums / jnp.maximum(cnt, 1.0)
    col = lax.broadcasted_iota(jnp.int32, (G, 128), 1)
    masked = jnp.where(col < C, pooled, -jnp.inf)
    m = jnp.max(masked, axis=1, keepdims=True)
    ex = jnp.where(col < C, jnp.exp(masked - m), 0.0)
    lse = jnp.log(jnp.sum(ex, axis=1, keepdims=True))
    o_ref[...] = masked - m - lse


def _sds(shape):
    return jax.ShapeDtypeStruct(shape, f32)


def kernel(x, edge_index, batch, mlp_W1, mlp_b1, mlp_W2, mlp_b2,
           Wrel1, brel1, Wroot1, Wrel2, brel2, Wroot2, Wrel3, brel3, Wroot3):
    src = edge_index[0]
    dst = edge_index[1]
    npad = EPAD - EDGES
    src_p = jnp.concatenate([src, jnp.zeros((npad,), jnp.int32)])
    dst_p = jnp.concatenate([dst, jnp.full((npad,), DUMMY_ROW, jnp.int32)])
    eidx = jnp.stack([src_p.reshape(-1, CHUNK), dst_p.reshape(-1, CHUNK)], axis=1)
    zeros128 = jnp.zeros((CHUNK, 128), f32)

    h1 = pl.pallas_call(_mlp_body, out_shape=_sds((NODES, D)))(
        x, mlp_W1, mlp_b1.reshape(1, -1), mlp_W2, mlp_b2.reshape(1, -1))

    p1 = _make_edge_agg(128)(h1, eidx, zeros128)
    oa1, ob1 = pl.pallas_call(
        _comb1_body, out_shape=(_sds((NODES, 128)), _sds((NODES, 128))))(
        p1, h1, Wrel1, brel1.reshape(1, -1), Wroot1)

    p2a = _make_edge_agg(128)(oa1, eidx, zeros128)
    p2b = _make_edge_agg(128)(ob1, eidx, zeros128)
    oa2, ob2 = pl.pallas_call(
        _comb2_body, out_shape=(_sds((NODES, 128)), _sds((NODES, 128))))(
        p2a, p2b, oa1, ob1, Wrel2, brel2.reshape(1, -1), Wroot2)

    Wrel3p = jnp.pad(Wrel3, ((0, 0), (0, 128 - C)))
    Wroot3p = jnp.pad(Wroot3, ((0, 0), (0, 128 - C)))
    brel3p = jnp.pad(brel3, (0, 128 - C)).reshape(1, -1)
    z, r = pl.pallas_call(
        _pre3_body, out_shape=(_sds((NODES, 128)), _sds((NODES, 128))))(
        oa2, ob2, Wrel3p, Wroot3p)
    p3 = _make_edge_agg(128)(z, eidx, zeros128)

    out = pl.pallas_call(_final_body, out_shape=_sds((G, 128)))(
        p3, r, brel3p, batch.reshape(-1, 1))
    return out[:, :C]


# speedup vs baseline: 1.1592x; 1.0630x over previous
import functools

import jax
import jax.numpy as jnp
from jax import lax
from jax.experimental import pallas as pl
from jax.experimental.pallas import tpu as pltpu
from jax.experimental.pallas import tpu_sc as plsc

NODES = 10000
EDGES = 320000
D = 128
H = 256
C = 10
G = 64

CHUNK = 128
NWORKERS = 32
CPW = 79
EPAD = NWORKERS * CPW * CHUNK
RPAD = 10112
RCHUNKS = RPAD // CHUNK
DUMMY_ROW = NODES

f32 = jnp.float32


@functools.lru_cache(maxsize=None)
def _make_edge_agg(width):
    mesh = plsc.VectorSubcoreMesh(core_axis_name="c", subcore_axis_name="s")

    @functools.partial(
        pl.kernel,
        mesh=mesh,
        out_type=jax.ShapeDtypeStruct((2, RPAD, width), f32),
        scratch_types=[
            pltpu.VMEM((2, CHUNK), jnp.int32),
            pltpu.VMEM((2, CHUNK), jnp.int32),
            pltpu.VMEM((CHUNK, width), f32),
            pltpu.VMEM((CHUNK, width), f32),
            pltpu.VMEM_SHARED((RPAD, width), f32),
            pltpu.SemaphoreType.DMA,
            pltpu.SemaphoreType.DMA,
            pltpu.SemaphoreType.DMA,
        ],
    )
    def agg(h_hbm, eidx_hbm, zeros_hbm, out_hbm, ebufa, ebufb, rows0, rows1,
            acc, sem0, sem1, sem2):
        cid = lax.axis_index("c")
        sid = lax.axis_index("s")
        wid = sid * 2 + cid

        pltpu.sync_copy(zeros_hbm, rows0)

        def zero_body(k, carry):
            chunk = sid + k * 16

            @pl.when(chunk < RCHUNKS)
            def _():
                off = pl.multiple_of(chunk * CHUNK, CHUNK)
                pltpu.sync_copy(rows0, acc.at[pl.ds(off, CHUNK)])

            return carry

        lax.fori_loop(0, (RCHUNKS + 15) // 16, zero_body, 0)
        plsc.subcore_barrier()

        ebase = wid * CPW
        nrows = NWORKERS * CPW

        def chunk(ebuf):
            pltpu.async_copy(h_hbm.at[ebuf.at[0]], rows0, sem0).wait()
            pltpu.sync_copy(rows0, acc.at[ebuf.at[1]], add=True)

        pltpu.sync_copy(eidx_hbm.at[ebase], ebufa)

        def edge_body(k, carry):
            a = k * 2
            n1 = pltpu.async_copy(
                eidx_hbm.at[jnp.minimum(ebase + a + 1, nrows - 1)], ebufb,
                sem1)
            pltpu.async_copy(h_hbm.at[ebufa.at[0]], rows0, sem0).wait()
            sa = pltpu.async_copy(rows0, acc.at[ebufa.at[1]], sem2, add=True)
            n1.wait()
            pltpu.async_copy(h_hbm.at[ebufb.at[0]], rows1, sem0).wait()
            sa.wait()
            n2 = pltpu.async_copy(
                eidx_hbm.at[jnp.minimum(ebase + a + 2, nrows - 1)], ebufa,
                sem1)
            pltpu.sync_copy(rows1, acc.at[ebufb.at[1]], add=True)
            n2.wait()
            return carry

        lax.fori_loop(0, CPW // 2, edge_body, 0)
        if CPW % 2:
            chunk(ebufa)
        plsc.subcore_barrier()

        def out_body(k, carry):
            chunk = sid + k * 16

            @pl.when(chunk < RCHUNKS)
            def _():
                off = pl.multiple_of(chunk * CHUNK, CHUNK)
                pltpu.sync_copy(acc.at[pl.ds(off, CHUNK)], rows0)
                pltpu.sync_copy(rows0, out_hbm.at[cid, pl.ds(off, CHUNK)])

            return carry

        lax.fori_loop(0, (RCHUNKS + 15) // 16, out_body, 0)

    return agg


def _mlp_body(x_ref, w1_ref, b1_ref, w2_ref, b2_ref, o_ref):
    h = jnp.dot(x_ref[...], w1_ref[...], preferred_element_type=f32)
    h = jnp.maximum(h + b1_ref[...], 0.0)
    h = jnp.dot(h, w2_ref[...], preferred_element_type=f32)
    o_ref[...] = jnp.maximum(h + b2_ref[...], 0.0)


def _comb1_body(p_ref, h_ref, wrel_ref, brel_ref, wroot_ref, oa_ref, ob_ref):
    agg = p_ref[0, :NODES, :] + p_ref[1, :NODES, :]
    out = (jnp.dot(agg, wrel_ref[...], preferred_element_type=f32)
           + brel_ref[...]
           + jnp.dot(h_ref[...], wroot_ref[...], preferred_element_type=f32))
    out = jnp.maximum(out, 0.0)
    oa_ref[...] = out[:, :128]
    ob_ref[...] = out[:, 128:]


def _comb2_body(pa_ref, pb_ref, ha_ref, hb_ref, wrel_ref, brel_ref,
                wroot_ref, oa_ref, ob_ref):
    agg_a = pa_ref[0, :NODES, :] + pa_ref[1, :NODES, :]
    agg_b = pb_ref[0, :NODES, :] + pb_ref[1, :NODES, :]
    out = (jnp.dot(agg_a, wrel_ref[:128, :], preferred_element_type=f32)
           + jnp.dot(agg_b, wrel_ref[128:, :], preferred_element_type=f32)
           + brel_ref[...]
           + jnp.dot(ha_ref[...], wroot_ref[:128, :], preferred_element_type=f32)
           + jnp.dot(hb_ref[...], wroot_ref[128:, :], preferred_element_type=f32))
    out = jnp.maximum(out, 0.0)
    oa_ref[...] = out[:, :128]
    ob_ref[...] = out[:, 128:]


def _pre3_body(ha_ref, hb_ref, wrel_ref, wroot_ref, z_ref, r_ref):
    z_ref[...] = (jnp.dot(ha_ref[...], wrel_ref[:128, :], preferred_element_type=f32)
                  + jnp.dot(hb_ref[...], wrel_ref[128:, :], preferred_element_type=f32))
    r_ref[...] = (jnp.dot(ha_ref[...], wroot_ref[:128, :], preferred_element_type=f32)
                  + jnp.dot(hb_ref[...], wroot_ref[128:, :], preferred_element_type=f32))


def _final_body(p_ref, r_ref, brel_ref, batch_ref, o_ref):
    out3 = (p_ref[0, :NODES, :] + p_ref[1, :NODES, :]
            + r_ref[...] + brel_ref[...])
    gids = lax.broadcasted_iota(jnp.int32, (NODES, G), 1)
    onehot = (batch_ref[...] == gids).astype(f32)
    sums = lax.dot_general(onehot, out3, (((0,), (0,)), ((), ())),
                           preferred_element_type=f32)
    cnt = jnp.sum(onehot, axis=0)[:, None]
    pooled = sums / jnp.maximum(cnt, 1.0)
    col = lax.broadcasted_iota(jnp.int32, (G, 128), 1)
    masked = jnp.where(col < C, pooled, -jnp.inf)
    m = jnp.max(masked, axis=1, keepdims=True)
    ex = jnp.where(col < C, jnp.exp(masked - m), 0.0)
    lse = jnp.log(jnp.sum(ex, axis=1, keepdims=True))
    o_ref[...] = masked - m - lse


def _sds(shape):
    return jax.ShapeDtypeStruct(shape, f32)


def kernel(x, edge_index, batch, mlp_W1, mlp_b1, mlp_W2, mlp_b2,
           Wrel1, brel1, Wroot1, Wrel2, brel2, Wroot2, Wrel3, brel3, Wroot3):
    src = edge_index[0]
    dst = edge_index[1]
    npad = EPAD - EDGES
    src_p = jnp.concatenate([src, jnp.zeros((npad,), jnp.int32)])
    dst_p = jnp.concatenate([dst, jnp.full((npad,), DUMMY_ROW, jnp.int32)])
    eidx = jnp.stack([src_p.reshape(-1, CHUNK), dst_p.reshape(-1, CHUNK)], axis=1)
    zeros128 = jnp.zeros((CHUNK, 128), f32)

    h1 = pl.pallas_call(_mlp_body, out_shape=_sds((NODES, D)))(
        x, mlp_W1, mlp_b1.reshape(1, -1), mlp_W2, mlp_b2.reshape(1, -1))

    p1 = _make_edge_agg(128)(h1, eidx, zeros128)
    oa1, ob1 = pl.pallas_call(
        _comb1_body, out_shape=(_sds((NODES, 128)), _sds((NODES, 128))))(
        p1, h1, Wrel1, brel1.reshape(1, -1), Wroot1)

    p2a = _make_edge_agg(128)(oa1, eidx, zeros128)
    p2b = _make_edge_agg(128)(ob1, eidx, zeros128)
    oa2, ob2 = pl.pallas_call(
        _comb2_body, out_shape=(_sds((NODES, 128)), _sds((NODES, 128))))(
        p2a, p2b, oa1, ob1, Wrel2, brel2.reshape(1, -1), Wroot2)

    Wrel3p = jnp.pad(Wrel3, ((0, 0), (0, 128 - C)))
    Wroot3p = jnp.pad(Wroot3, ((0, 0), (0, 128 - C)))
    brel3p = jnp.pad(brel3, (0, 128 - C)).reshape(1, -1)
    z, r = pl.pallas_call(
        _pre3_body, out_shape=(_sds((NODES, 128)), _sds((NODES, 128))))(
        oa2, ob2, Wrel3p, Wroot3p)
    p3 = _make_edge_agg(128)(z, eidx, zeros128)

    out = pl.pallas_call(_final_body, out_shape=_sds((G, 128)))(
        p3, r, brel3p, batch.reshape(-1, 1))
    return out[:, :C]
